# baseline (speedup 1.0000x reference)
import functools

import jax
import jax.numpy as jnp
import numpy as np
from jax import lax
from jax.experimental import pallas as pl
from jax.experimental.pallas import tpu as pltpu
from jax.experimental.pallas import tpu_sc as plsc

N = 10000
NPAD = 10112
NC = 2
NS = 16
LG = 128
NW = NC * NS
E_TOT = 320000 + N
NR = -(-E_TOT // (NW * LG))
NR = NR + (NR % 2)
E_PAD = NW * NR * LG

BM = 1000


def _sc_degree(dst_r, zeros16, ones16):
    mesh = plsc.VectorSubcoreMesh(core_axis_name="c", subcore_axis_name="s")

    @functools.partial(
        pl.kernel,
        out_type=jax.ShapeDtypeStruct((NC, NPAD, 16), jnp.float32),
        mesh=mesh,
        scratch_types=[
            pltpu.VMEM((NR, 128), jnp.int32),
            pltpu.VMEM((128, 16), jnp.float32),
            pltpu.VMEM_SHARED((NPAD, 16), jnp.float32),
        ],
    )
    def k(dst_hbm, zero_hbm, ones_hbm, out_hbm, dst_v, ones_v, acc):
        c = lax.axis_index("c")
        s = lax.axis_index("s")
        wid = c * NS + s
        pltpu.sync_copy(dst_hbm.at[wid], dst_v)
        pltpu.sync_copy(ones_hbm, ones_v)
        rz = NPAD // NS
        pltpu.sync_copy(zero_hbm.at[pl.ds(s * rz, rz)], acc.at[pl.ds(s * rz, rz)])
        plsc.subcore_barrier()

        @pl.loop(0, NR)
        def _(r):
            pltpu.sync_copy(ones_v, acc.at[dst_v.at[r]], add=True)

        plsc.subcore_barrier()
        pltpu.sync_copy(acc.at[pl.ds(s * rz, rz)],
                        out_hbm.at[c, pl.ds(s * rz, rz)])

    return k(dst_r, zeros16, ones16)


_SC_KERNEL_CACHE = {}


def _sc_aggregate(xs, idx_p, zeros, feat):
    if feat in _SC_KERNEL_CACHE:
        return _SC_KERNEL_CACHE[feat](xs, idx_p, zeros)

    mesh = plsc.VectorSubcoreMesh(core_axis_name="c", subcore_axis_name="s")

    @functools.partial(
        pl.kernel,
        out_type=jax.ShapeDtypeStruct((NC, NPAD, feat), jnp.float32),
        mesh=mesh,
        scratch_types=[
            pltpu.VMEM((NR, 128), jnp.int32),
            pltpu.VMEM((1, 128), jnp.int32),
            pltpu.VMEM((1, 128), jnp.int32),
            pltpu.VMEM((1, 128), jnp.int32),
            pltpu.VMEM((1, 128), jnp.int32),
            pltpu.VMEM((LG, feat), jnp.float32),
            pltpu.VMEM((LG, feat), jnp.float32),
            pltpu.VMEM_SHARED((NPAD, feat), jnp.float32),
            pltpu.SemaphoreType.DMA,
            pltpu.SemaphoreType.DMA,
            pltpu.SemaphoreType.DMA,
            pltpu.SemaphoreType.DMA,
        ],
    )
    def k(xs_hbm, idx_hbm, zero_hbm, out_hbm,
          idx_v, sia, sib, dia, dib, buf_a, buf_b, acc,
          sem_a, sem_b, sem_sa, sem_sb):
        c = lax.axis_index("c")
        s = lax.axis_index("s")
        wid = c * NS + s
        pltpu.sync_copy(idx_hbm.at[wid], idx_v)
        rz = NPAD // NS
        pltpu.sync_copy(zero_hbm.at[pl.ds(s * rz, rz)], acc.at[pl.ds(s * rz, rz)])
        plsc.subcore_barrier()

        def unpack(r, si, di):
            for l in range(8):
                w = idx_v[r, pl.ds(l * 16, 16)]
                si[0, pl.ds(l * 16, 16)] = w & 0xFFFF
                di[0, pl.ds(l * 16, 16)] = lax.shift_right_logical(w, 16)

        def start(si, buf, sem):
            pltpu.async_copy(xs_hbm.at[si.at[0]], buf, sem)

        def wait(buf, sem):
            pltpu.make_async_copy(zero_hbm.at[pl.ds(0, LG)], buf, sem).wait()

        def scat_start(di, buf, sem):
            pltpu.async_copy(buf, acc.at[di.at[0]], sem, add=True)

        def scat_wait(buf, sem):
            pltpu.make_async_copy(buf, acc.at[pl.ds(0, LG)], sem).wait()

        unpack(0, sia, dia)
        start(sia, buf_a, sem_a)

        @pl.loop(0, NR, step=2)
        def _(r):
            wait(buf_a, sem_a)
            scat_start(dia, buf_a, sem_sa)

            @pl.when(r > 0)
            def _():
                scat_wait(buf_b, sem_sb)

            unpack(r + 1, sib, dib)
            start(sib, buf_b, sem_b)
            wait(buf_b, sem_b)
            scat_start(dib, buf_b, sem_sb)
            scat_wait(buf_a, sem_sa)

            @pl.when(r + 2 < NR)
            def _():
                unpack(r + 2, sia, dia)
                start(sia, buf_a, sem_a)

        scat_wait(buf_b, sem_sb)
        plsc.subcore_barrier()
        pltpu.sync_copy(acc.at[pl.ds(s * rz, rz)],
                        out_hbm.at[c, pl.ds(s * rz, rz)])

    _SC_KERNEL_CACHE[feat] = k
    return k(xs, idx_p, zeros)


def _tc_prescale(degp, x):
    def body(degp_ref, x_ref, dinv_ref, xs_ref):
        d = degp_ref[0] + degp_ref[1]
        dinv = lax.rsqrt(d[:, 0:1])
        dinv_ref[...] = jnp.broadcast_to(dinv, (BM, 128))
        xs_ref[...] = x_ref[...] * dinv

    return pl.pallas_call(
        body,
        grid=(N // BM,),
        in_specs=[pl.BlockSpec((NC, BM, 16), lambda i: (0, i, 0)),
                  pl.BlockSpec((BM, 128), lambda i: (i, 0))],
        out_specs=[pl.BlockSpec((BM, 128), lambda i: (i, 0)),
                   pl.BlockSpec((BM, 128), lambda i: (i, 0))],
        out_shape=[jax.ShapeDtypeStruct((N, 128), jnp.float32),
                   jax.ShapeDtypeStruct((N, 128), jnp.float32)],
    )(degp, x)


def _mm(a, w):
    return lax.dot_general(a, w, (((1,), (0,)), ((), ())),
                           preferred_element_type=jnp.float32)


def _tc_layer1(p, dinv, W1, b1, W2):
    def body(p_ref, dinv_ref, w1_ref, b1_ref, w2_ref, o_ref):
        dv = dinv_ref[...]
        t = (p_ref[0] + p_ref[1]) * dv
        h = jnp.maximum(_mm(t, w1_ref[...]) + b1_ref[...], 0.0)
        o_ref[...] = _mm(h, w2_ref[...]) * dv

    return pl.pallas_call(
        body,
        grid=(N // BM,),
        in_specs=[pl.BlockSpec((NC, BM, 128), lambda i: (0, i, 0)),
                  pl.BlockSpec((BM, 128), lambda i: (i, 0)),
                  pl.BlockSpec((128, 256), lambda i: (0, 0)),
                  pl.BlockSpec((1, 256), lambda i: (0, 0)),
                  pl.BlockSpec((256, 128), lambda i: (0, 0))],
        out_specs=pl.BlockSpec((BM, 128), lambda i: (i, 0)),
        out_shape=jax.ShapeDtypeStruct((N, 128), jnp.float32),
    )(p, dinv, W1, b1, W2)


def _tc_layer2(p, dinv, b2, W3p):
    def body(p_ref, dinv_ref, b2_ref, w3_ref, o_ref):
        dv = dinv_ref[...]
        h = jnp.maximum((p_ref[0] + p_ref[1]) * dv + b2_ref[...], 0.0)
        o_ref[...] = _mm(h, w3_ref[...]) * dv

    return pl.pallas_call(
        body,
        grid=(N // BM,),
        in_specs=[pl.BlockSpec((NC, BM, 128), lambda i: (0, i, 0)),
                  pl.BlockSpec((BM, 128), lambda i: (i, 0)),
                  pl.BlockSpec((1, 128), lambda i: (0, 0)),
                  pl.BlockSpec((128, 128), lambda i: (0, 0))],
        out_specs=pl.BlockSpec((BM, 128), lambda i: (i, 0)),
        out_shape=jax.ShapeDtypeStruct((N, 128), jnp.float32),
    )(p, dinv, b2, W3p)


def _tc_layer3(p, dinv, b3):
    def body(p_ref, dinv_ref, b3_ref, o_ref):
        t = ((p_ref[0] + p_ref[1]) * dinv_ref[...])[:, 0:64] + b3_ref[...]
        m = jnp.max(t, axis=1, keepdims=True)
        e = jnp.exp(t - m)
        o_ref[...] = t - m - jnp.log(jnp.sum(e, axis=1, keepdims=True))

    return pl.pallas_call(
        body,
        grid=(N // BM,),
        in_specs=[pl.BlockSpec((NC, BM, 128), lambda i: (0, i, 0)),
                  pl.BlockSpec((BM, 128), lambda i: (i, 0)),
                  pl.BlockSpec((1, 64), lambda i: (0, 0))],
        out_specs=pl.BlockSpec((BM, 64), lambda i: (i, 0)),
        out_shape=jax.ShapeDtypeStruct((N, 64), jnp.float32),
    )(p, dinv, b3)


def kernel(x, edge_index, W1, b1, W2, b2, W3, b3):
    ei = edge_index.astype(jnp.int32)
    pad = E_PAD - E_TOT
    loop = np.arange(N, dtype=np.int32)
    pad_src = np.arange(pad, dtype=np.int32) * 997 % N
    pad_dst = N + np.arange(pad, dtype=np.int32) % (NPAD - N)
    src = jnp.concatenate([ei[0], jnp.asarray(np.concatenate([loop, pad_src]))])
    dst = jnp.concatenate([ei[1], jnp.asarray(np.concatenate([loop, pad_dst]))])
    idx_p = (src | (dst << 16)).reshape(NW, NR, 128)
    dst_r = dst.reshape(NW, NR, 128)

    zeros128 = jnp.asarray(np.zeros((NPAD, 128), np.float32))
    zeros16 = jnp.asarray(np.zeros((NPAD, 16), np.float32))
    ones16 = jnp.asarray(np.ones((128, 16), np.float32))
    W3p = jnp.pad(W3, ((0, 0), (0, 64)))

    degp = _sc_degree(dst_r, zeros16, ones16)
    dinv, xs = _tc_prescale(degp, x)
    p1 = _sc_aggregate(xs, idx_p, zeros128, 128)
    u2 = _tc_layer1(p1, dinv, W1, b1.reshape(1, -1), W2)
    p2 = _sc_aggregate(u2, idx_p, zeros128, 128)
    u3 = _tc_layer2(p2, dinv, b2.reshape(1, -1), W3p)
    p3 = _sc_aggregate(u3, idx_p, zeros128, 128)
    return _tc_layer3(p3, dinv, b3.reshape(1, -1))


# speedup vs baseline: 1.1696x; 1.1696x over previous
import functools

import jax
import jax.numpy as jnp
import numpy as np
from jax import lax
from jax.experimental import pallas as pl
from jax.experimental.pallas import tpu as pltpu
from jax.experimental.pallas import tpu_sc as plsc

N = 10000
NPAD = 10112
NC = 2
NS = 16
LG = 128
NW = NC * NS
E_TOT = 320000 + N
NR = -(-E_TOT // (NW * LG))
NR = NR + (NR % 2)
E_PAD = NW * NR * LG

BM = 1000


def _sc_degree(dst_r, zeros16, ones16):
    mesh = plsc.VectorSubcoreMesh(core_axis_name="c", subcore_axis_name="s")

    @functools.partial(
        pl.kernel,
        out_type=jax.ShapeDtypeStruct((NC, NPAD, 16), jnp.float32),
        mesh=mesh,
        scratch_types=[
            pltpu.VMEM((NR, 128), jnp.int32),
            pltpu.VMEM((128, 16), jnp.float32),
            pltpu.VMEM_SHARED((NPAD, 16), jnp.float32),
        ],
    )
    def k(dst_hbm, zero_hbm, ones_hbm, out_hbm, dst_v, ones_v, acc):
        c = lax.axis_index("c")
        s = lax.axis_index("s")
        wid = c * NS + s
        pltpu.sync_copy(dst_hbm.at[wid], dst_v)
        pltpu.sync_copy(ones_hbm, ones_v)
        rz = NPAD // NS
        pltpu.sync_copy(zero_hbm.at[pl.ds(s * rz, rz)], acc.at[pl.ds(s * rz, rz)])
        plsc.subcore_barrier()

        @pl.loop(0, NR)
        def _(r):
            pltpu.sync_copy(ones_v, acc.at[dst_v.at[r]], add=True)

        plsc.subcore_barrier()
        pltpu.sync_copy(acc.at[pl.ds(s * rz, rz)],
                        out_hbm.at[c, pl.ds(s * rz, rz)])

    return k(dst_r, zeros16, ones16)


_SC_KERNEL_CACHE = {}


def _sc_aggregate(xs, idx_p, zeros, feat):
    if feat in _SC_KERNEL_CACHE:
        return _SC_KERNEL_CACHE[feat](xs, idx_p, zeros)

    mesh = plsc.VectorSubcoreMesh(core_axis_name="c", subcore_axis_name="s")

    @functools.partial(
        pl.kernel,
        out_type=jax.ShapeDtypeStruct((NC, NPAD, feat), jnp.float32),
        mesh=mesh,
        scratch_types=[
            pltpu.VMEM((NR, 128), jnp.int32),
            pltpu.VMEM((1, 128), jnp.int32),
            pltpu.VMEM((1, 128), jnp.int32),
            pltpu.VMEM((1, 128), jnp.int32),
            pltpu.VMEM((1, 128), jnp.int32),
            pltpu.VMEM((LG, feat), jnp.float32),
            pltpu.VMEM((LG, feat), jnp.float32),
            pltpu.VMEM_SHARED((NPAD, feat), jnp.float32),
            pltpu.SemaphoreType.DMA,
            pltpu.SemaphoreType.DMA,
        ],
    )
    def k(xs_hbm, idx_hbm, zero_hbm, out_hbm,
          idx_v, sia, sib, dia, dib, buf_a, buf_b, acc,
          sem_a, sem_b):
        c = lax.axis_index("c")
        s = lax.axis_index("s")
        wid = c * NS + s
        pltpu.sync_copy(idx_hbm.at[wid], idx_v)
        rz = NPAD // NS
        pltpu.sync_copy(zero_hbm.at[pl.ds(s * rz, rz)], acc.at[pl.ds(s * rz, rz)])
        plsc.subcore_barrier()

        def unpack(r, si, di):
            for l in range(8):
                w = idx_v[r, pl.ds(l * 16, 16)]
                si[0, pl.ds(l * 16, 16)] = w & 0xFFFF
                di[0, pl.ds(l * 16, 16)] = lax.shift_right_logical(w, 16)

        def start(si, buf, sem):
            pltpu.async_copy(xs_hbm.at[si.at[0]], buf, sem)

        def wait(buf, sem):
            pltpu.make_async_copy(zero_hbm.at[pl.ds(0, LG)], buf, sem).wait()

        def scat(di, buf):
            pltpu.sync_copy(buf, acc.at[di.at[0]], add=True)

        unpack(0, sia, dia)
        start(sia, buf_a, sem_a)

        @pl.loop(0, NR, step=2)
        def _(r):
            unpack(r + 1, sib, dib)
            start(sib, buf_b, sem_b)
            wait(buf_a, sem_a)
            scat(dia, buf_a)

            @pl.when(r + 2 < NR)
            def _():
                unpack(r + 2, sia, dia)
                start(sia, buf_a, sem_a)

            wait(buf_b, sem_b)
            scat(dib, buf_b)

        plsc.subcore_barrier()
        pltpu.sync_copy(acc.at[pl.ds(s * rz, rz)],
                        out_hbm.at[c, pl.ds(s * rz, rz)])

    _SC_KERNEL_CACHE[feat] = k
    return k(xs, idx_p, zeros)


def _tc_prescale(degp, x):
    def body(degp_ref, x_ref, dinv_ref, xs_ref):
        d = degp_ref[0] + degp_ref[1]
        dinv = lax.rsqrt(d[:, 0:1])
        dinv_ref[...] = jnp.broadcast_to(dinv, (BM, 128))
        xs_ref[...] = x_ref[...] * dinv

    return pl.pallas_call(
        body,
        grid=(N // BM,),
        in_specs=[pl.BlockSpec((NC, BM, 16), lambda i: (0, i, 0)),
                  pl.BlockSpec((BM, 128), lambda i: (i, 0))],
        out_specs=[pl.BlockSpec((BM, 128), lambda i: (i, 0)),
                   pl.BlockSpec((BM, 128), lambda i: (i, 0))],
        out_shape=[jax.ShapeDtypeStruct((N, 128), jnp.float32),
                   jax.ShapeDtypeStruct((N, 128), jnp.float32)],
    )(degp, x)


def _mm(a, w):
    return lax.dot_general(a, w, (((1,), (0,)), ((), ())),
                           preferred_element_type=jnp.float32)


def _tc_layer1(p, dinv, W1, b1, W2):
    def body(p_ref, dinv_ref, w1_ref, b1_ref, w2_ref, o_ref):
        dv = dinv_ref[...]
        t = (p_ref[0] + p_ref[1]) * dv
        h = jnp.maximum(_mm(t, w1_ref[...]) + b1_ref[...], 0.0)
        o_ref[...] = _mm(h, w2_ref[...]) * dv

    return pl.pallas_call(
        body,
        grid=(N // BM,),
        in_specs=[pl.BlockSpec((NC, BM, 128), lambda i: (0, i, 0)),
                  pl.BlockSpec((BM, 128), lambda i: (i, 0)),
                  pl.BlockSpec((128, 256), lambda i: (0, 0)),
                  pl.BlockSpec((1, 256), lambda i: (0, 0)),
                  pl.BlockSpec((256, 128), lambda i: (0, 0))],
        out_specs=pl.BlockSpec((BM, 128), lambda i: (i, 0)),
        out_shape=jax.ShapeDtypeStruct((N, 128), jnp.float32),
    )(p, dinv, W1, b1, W2)


def _tc_layer2(p, dinv, b2, W3p):
    def body(p_ref, dinv_ref, b2_ref, w3_ref, o_ref):
        dv = dinv_ref[...]
        h = jnp.maximum((p_ref[0] + p_ref[1]) * dv + b2_ref[...], 0.0)
        o_ref[...] = _mm(h, w3_ref[...]) * dv

    return pl.pallas_call(
        body,
        grid=(N // BM,),
        in_specs=[pl.BlockSpec((NC, BM, 128), lambda i: (0, i, 0)),
                  pl.BlockSpec((BM, 128), lambda i: (i, 0)),
                  pl.BlockSpec((1, 128), lambda i: (0, 0)),
                  pl.BlockSpec((128, 128), lambda i: (0, 0))],
        out_specs=pl.BlockSpec((BM, 128), lambda i: (i, 0)),
        out_shape=jax.ShapeDtypeStruct((N, 128), jnp.float32),
    )(p, dinv, b2, W3p)


def _tc_layer3(p, dinv, b3):
    def body(p_ref, dinv_ref, b3_ref, o_ref):
        t = ((p_ref[0] + p_ref[1]) * dinv_ref[...])[:, 0:64] + b3_ref[...]
        m = jnp.max(t, axis=1, keepdims=True)
        e = jnp.exp(t - m)
        o_ref[...] = t - m - jnp.log(jnp.sum(e, axis=1, keepdims=True))

    return pl.pallas_call(
        body,
        grid=(N // BM,),
        in_specs=[pl.BlockSpec((NC, BM, 128), lambda i: (0, i, 0)),
                  pl.BlockSpec((BM, 128), lambda i: (i, 0)),
                  pl.BlockSpec((1, 64), lambda i: (0, 0))],
        out_specs=pl.BlockSpec((BM, 64), lambda i: (i, 0)),
        out_shape=jax.ShapeDtypeStruct((N, 64), jnp.float32),
    )(p, dinv, b3)


def kernel(x, edge_index, W1, b1, W2, b2, W3, b3):
    ei = edge_index.astype(jnp.int32)
    pad = E_PAD - E_TOT
    loop = np.arange(N, dtype=np.int32)
    pad_src = np.arange(pad, dtype=np.int32) * 997 % N
    pad_dst = N + np.arange(pad, dtype=np.int32) % (NPAD - N)
    src = jnp.concatenate([ei[0], jnp.asarray(np.concatenate([loop, pad_src]))])
    dst = jnp.concatenate([ei[1], jnp.asarray(np.concatenate([loop, pad_dst]))])
    idx_p = (src | (dst << 16)).reshape(NW, NR, 128)
    dst_r = dst.reshape(NW, NR, 128)

    zeros128 = jnp.asarray(np.zeros((NPAD, 128), np.float32))
    zeros16 = jnp.asarray(np.zeros((NPAD, 16), np.float32))
    ones16 = jnp.asarray(np.ones((128, 16), np.float32))
    W3p = jnp.pad(W3, ((0, 0), (0, 64)))

    degp = _sc_degree(dst_r, zeros16, ones16)
    dinv, xs = _tc_prescale(degp, x)
    p1 = _sc_aggregate(xs, idx_p, zeros128, 128)
    u2 = _tc_layer1(p1, dinv, W1, b1.reshape(1, -1), W2)
    p2 = _sc_aggregate(u2, idx_p, zeros128, 128)
    u3 = _tc_layer2(p2, dinv, b2.reshape(1, -1), W3p)
    p3 = _sc_aggregate(u3, idx_p, zeros128, 128)
    return _tc_layer3(p3, dinv, b3.reshape(1, -1))


# speedup vs baseline: 1.1712x; 1.0014x over previous
import functools

import jax
import jax.numpy as jnp
import numpy as np
from jax import lax
from jax.experimental import pallas as pl
from jax.experimental.pallas import tpu as pltpu
from jax.experimental.pallas import tpu_sc as plsc

N = 10000
NPAD = 10112
NC = 2
NS = 16
LG = 128
NW = NC * NS
E_TOT = 320000 + N
NR = -(-E_TOT // (NW * LG))
NR = NR + (NR % 2)
E_PAD = NW * NR * LG

BM = 1000


def _sc_degree(dst_r, zeros16, ones16):
    mesh = plsc.VectorSubcoreMesh(core_axis_name="c", subcore_axis_name="s")

    @functools.partial(
        pl.kernel,
        out_type=jax.ShapeDtypeStruct((NC, NPAD, 16), jnp.float32),
        mesh=mesh,
        scratch_types=[
            pltpu.VMEM((NR, 128), jnp.int32),
            pltpu.VMEM((128, 16), jnp.float32),
            pltpu.VMEM_SHARED((NPAD, 16), jnp.float32),
        ],
    )
    def k(dst_hbm, zero_hbm, ones_hbm, out_hbm, dst_v, ones_v, acc):
        c = lax.axis_index("c")
        s = lax.axis_index("s")
        wid = c * NS + s
        pltpu.sync_copy(dst_hbm.at[wid], dst_v)
        pltpu.sync_copy(ones_hbm, ones_v)
        rz = NPAD // NS
        pltpu.sync_copy(zero_hbm.at[pl.ds(s * rz, rz)], acc.at[pl.ds(s * rz, rz)])
        plsc.subcore_barrier()

        @pl.loop(0, NR)
        def _(r):
            pltpu.sync_copy(ones_v, acc.at[dst_v.at[r]], add=True)

        plsc.subcore_barrier()
        pltpu.sync_copy(acc.at[pl.ds(s * rz, rz)],
                        out_hbm.at[c, pl.ds(s * rz, rz)])

    return k(dst_r, zeros16, ones16)


_SC_KERNEL_CACHE = {}


def _sc_aggregate(xs, idx_p, zeros, feat, feat_out=None):
    fo = feat if feat_out is None else feat_out
    if (feat, fo) in _SC_KERNEL_CACHE:
        return _SC_KERNEL_CACHE[(feat, fo)](xs, idx_p, zeros)

    mesh = plsc.VectorSubcoreMesh(core_axis_name="c", subcore_axis_name="s")

    @functools.partial(
        pl.kernel,
        out_type=jax.ShapeDtypeStruct((NC, NPAD, fo), jnp.float32),
        mesh=mesh,
        scratch_types=[
            pltpu.VMEM((NR, 128), jnp.int32),
            pltpu.VMEM((1, 128), jnp.int32),
            pltpu.VMEM((1, 128), jnp.int32),
            pltpu.VMEM((1, 128), jnp.int32),
            pltpu.VMEM((1, 128), jnp.int32),
            pltpu.VMEM((LG, feat), jnp.float32),
            pltpu.VMEM((LG, feat), jnp.float32),
            pltpu.VMEM_SHARED((NPAD, fo), jnp.float32),
            pltpu.SemaphoreType.DMA,
            pltpu.SemaphoreType.DMA,
        ],
    )
    def k(xs_hbm, idx_hbm, zero_hbm, out_hbm,
          idx_v, sia, sib, dia, dib, buf_a, buf_b, acc,
          sem_a, sem_b):
        c = lax.axis_index("c")
        s = lax.axis_index("s")
        wid = c * NS + s
        pltpu.sync_copy(idx_hbm.at[wid], idx_v)
        rz = NPAD // NS
        pltpu.sync_copy(zero_hbm.at[pl.ds(s * rz, rz)], acc.at[pl.ds(s * rz, rz)])
        plsc.subcore_barrier()

        def unpack(r, si, di):
            for l in range(8):
                w = idx_v[r, pl.ds(l * 16, 16)]
                si[0, pl.ds(l * 16, 16)] = w & 0xFFFF
                di[0, pl.ds(l * 16, 16)] = lax.shift_right_logical(w, 16)

        def start(si, buf, sem):
            pltpu.async_copy(xs_hbm.at[si.at[0]], buf, sem)

        def wait(buf, sem):
            pltpu.make_async_copy(xs_hbm.at[pl.ds(0, LG)], buf, sem).wait()

        def scat(di, buf):
            src = buf if fo == feat else buf.at[:, pl.ds(0, fo)]
            pltpu.sync_copy(src, acc.at[di.at[0]], add=True)

        unpack(0, sia, dia)
        start(sia, buf_a, sem_a)

        @pl.loop(0, NR, step=2)
        def _(r):
            unpack(r + 1, sib, dib)
            start(sib, buf_b, sem_b)
            wait(buf_a, sem_a)
            scat(dia, buf_a)

            @pl.when(r + 2 < NR)
            def _():
                unpack(r + 2, sia, dia)
                start(sia, buf_a, sem_a)

            wait(buf_b, sem_b)
            scat(dib, buf_b)

        plsc.subcore_barrier()
        pltpu.sync_copy(acc.at[pl.ds(s * rz, rz)],
                        out_hbm.at[c, pl.ds(s * rz, rz)])

    _SC_KERNEL_CACHE[(feat, fo)] = k
    return k(xs, idx_p, zeros)


def _tc_prescale(degp, x):
    def body(degp_ref, x_ref, dinv_ref, xs_ref):
        d = degp_ref[0] + degp_ref[1]
        dinv = lax.rsqrt(d[:, 0:1])
        dinv_ref[...] = jnp.broadcast_to(dinv, (BM, 128))
        xs_ref[...] = x_ref[...] * dinv

    return pl.pallas_call(
        body,
        grid=(N // BM,),
        in_specs=[pl.BlockSpec((NC, BM, 16), lambda i: (0, i, 0)),
                  pl.BlockSpec((BM, 128), lambda i: (i, 0))],
        out_specs=[pl.BlockSpec((BM, 128), lambda i: (i, 0)),
                   pl.BlockSpec((BM, 128), lambda i: (i, 0))],
        out_shape=[jax.ShapeDtypeStruct((N, 128), jnp.float32),
                   jax.ShapeDtypeStruct((N, 128), jnp.float32)],
    )(degp, x)


def _mm(a, w):
    return lax.dot_general(a, w, (((1,), (0,)), ((), ())),
                           preferred_element_type=jnp.float32)


def _tc_layer1(p, dinv, W1, b1, W2):
    def body(p_ref, dinv_ref, w1_ref, b1_ref, w2_ref, o_ref):
        dv = dinv_ref[...]
        t = (p_ref[0] + p_ref[1]) * dv
        h = jnp.maximum(_mm(t, w1_ref[...]) + b1_ref[...], 0.0)
        o_ref[...] = _mm(h, w2_ref[...]) * dv

    return pl.pallas_call(
        body,
        grid=(N // BM,),
        in_specs=[pl.BlockSpec((NC, BM, 128), lambda i: (0, i, 0)),
                  pl.BlockSpec((BM, 128), lambda i: (i, 0)),
                  pl.BlockSpec((128, 256), lambda i: (0, 0)),
                  pl.BlockSpec((1, 256), lambda i: (0, 0)),
                  pl.BlockSpec((256, 128), lambda i: (0, 0))],
        out_specs=pl.BlockSpec((BM, 128), lambda i: (i, 0)),
        out_shape=jax.ShapeDtypeStruct((N, 128), jnp.float32),
    )(p, dinv, W1, b1, W2)


def _tc_layer2(p, dinv, b2, W3p):
    def body(p_ref, dinv_ref, b2_ref, w3_ref, o_ref):
        dv = dinv_ref[...]
        h = jnp.maximum((p_ref[0] + p_ref[1]) * dv + b2_ref[...], 0.0)
        o_ref[...] = _mm(h, w3_ref[...]) * dv

    return pl.pallas_call(
        body,
        grid=(N // BM,),
        in_specs=[pl.BlockSpec((NC, BM, 128), lambda i: (0, i, 0)),
                  pl.BlockSpec((BM, 128), lambda i: (i, 0)),
                  pl.BlockSpec((1, 128), lambda i: (0, 0)),
                  pl.BlockSpec((128, 128), lambda i: (0, 0))],
        out_specs=pl.BlockSpec((BM, 128), lambda i: (i, 0)),
        out_shape=jax.ShapeDtypeStruct((N, 128), jnp.float32),
    )(p, dinv, b2, W3p)


def _tc_layer3(p, dinv, b3):
    def body(p_ref, dinv_ref, b3_ref, o_ref):
        t = ((p_ref[0] + p_ref[1]) * dinv_ref[...])[:, 0:64] + b3_ref[...]
        m = jnp.max(t, axis=1, keepdims=True)
        e = jnp.exp(t - m)
        o_ref[...] = t - m - jnp.log(jnp.sum(e, axis=1, keepdims=True))

    return pl.pallas_call(
        body,
        grid=(N // BM,),
        in_specs=[pl.BlockSpec((NC, BM, 128), lambda i: (0, i, 0)),
                  pl.BlockSpec((BM, 128), lambda i: (i, 0)),
                  pl.BlockSpec((1, 64), lambda i: (0, 0))],
        out_specs=pl.BlockSpec((BM, 64), lambda i: (i, 0)),
        out_shape=jax.ShapeDtypeStruct((N, 64), jnp.float32),
    )(p, dinv, b3)


def kernel(x, edge_index, W1, b1, W2, b2, W3, b3):
    ei = edge_index.astype(jnp.int32)
    pad = E_PAD - E_TOT
    loop = np.arange(N, dtype=np.int32)
    pad_src = np.arange(pad, dtype=np.int32) * 997 % N
    pad_dst = N + np.arange(pad, dtype=np.int32) % (NPAD - N)
    src = jnp.concatenate([ei[0], jnp.asarray(np.concatenate([loop, pad_src]))])
    dst = jnp.concatenate([ei[1], jnp.asarray(np.concatenate([loop, pad_dst]))])
    idx_p = (src | (dst << 16)).reshape(NW, NR, 128)
    dst_r = dst.reshape(NW, NR, 128)

    zeros128 = jnp.asarray(np.zeros((NPAD, 128), np.float32))
    zeros16 = jnp.asarray(np.zeros((NPAD, 16), np.float32))
    ones16 = jnp.asarray(np.ones((128, 16), np.float32))
    W3p = jnp.pad(W3, ((0, 0), (0, 64)))

    degp = _sc_degree(dst_r, zeros16, ones16)
    dinv, xs = _tc_prescale(degp, x)
    p1 = _sc_aggregate(xs, idx_p, zeros128, 128)
    u2 = _tc_layer1(p1, dinv, W1, b1.reshape(1, -1), W2)
    p2 = _sc_aggregate(u2, idx_p, zeros128, 128)
    u3 = _tc_layer2(p2, dinv, b2.reshape(1, -1), W3p)
    p3 = _sc_aggregate(u3, idx_p, zeros128, 128)
    return _tc_layer3(p3, dinv, b3.reshape(1, -1))
